# Initial kernel scaffold; baseline (speedup 1.0000x reference)
import jax
import jax.numpy as jnp
from jax.experimental import pallas as pl


def kernel(px, lx, pl_mat_list, W_pl, b_pl, g_pl, be_pl, W_lp, b_lp, g_lp, be_lp, wp, bp, wl, bl):
    raise NotImplementedError("write your pallas kernel here")


# speedup vs baseline: 2.5300x; 2.5300x over previous
import functools

import jax
import jax.numpy as jnp
from jax.experimental import pallas as pl
from jax.experimental.pallas import tpu as pltpu

N_EXP = 4
NP = 8192
NL = 2048
DP = 64
DL = 64
DO = 64
DIN = DP + DL
EPS = 1e-5

BLK = 1024
NR = NP // BLK


def _body(am_ref, px_ref, lx_ref,
          Wpl_ref, bpl_ref, gpl_ref, bepl_ref,
          Wlp_ref, blp_ref, glp_ref, belp_ref,
          wp_ref, bp_ref, wl_ref, bl_ref,
          px_out_ref, lx_out_ref,
          z2_ref, x1_ref):
    i = pl.program_id(0)
    r = pl.program_id(1)
    nr = pl.num_programs(1)

    am = am_ref[0]
    pxb = px_ref[...]
    lx = lx_ref[...]

    x2 = jnp.dot(am, lx, preferred_element_type=jnp.float32)
    Wpl = Wpl_ref[0]
    z2 = (jnp.dot(x2, Wpl[:DL], preferred_element_type=jnp.float32)
          + jnp.dot(pxb, Wpl[DL:], preferred_element_type=jnp.float32)
          + bpl_ref[0])
    z2_ref[pl.ds(r * BLK, BLK), :] = z2

    part = jax.lax.dot_general(am, pxb, (((0,), (0,)), ((), ())),
                               preferred_element_type=jnp.float32)

    @pl.when(r == 0)
    def _():
        x1_ref[...] = part

    @pl.when(r != 0)
    def _():
        x1_ref[...] = x1_ref[...] + part

    @pl.when(r == nr - 1)
    def _():
        Wlp = Wlp_ref[0]
        z1 = (jnp.dot(x1_ref[...], Wlp[:DP], preferred_element_type=jnp.float32)
              + jnp.dot(lx, Wlp[DP:], preferred_element_type=jnp.float32)
              + blp_ref[0])
        m1 = jnp.mean(z1, axis=0, keepdims=True)
        z1c = z1 - m1
        v1 = jnp.mean(z1c * z1c, axis=0, keepdims=True)
        p1 = jnp.maximum(z1c * jax.lax.rsqrt(v1 + EPS) * glp_ref[0]
                         + belp_ref[0], 0.0)
        contrib_l = wl_ref[0] * p1

        @pl.when(i == 0)
        def _():
            lx_out_ref[...] = contrib_l + bl_ref[0]

        @pl.when(i != 0)
        def _():
            lx_out_ref[...] = lx_out_ref[...] + contrib_l

        zb = z2_ref[...]
        m2 = jnp.mean(zb, axis=0, keepdims=True)
        zbc = zb - m2
        v2 = jnp.mean(zbc * zbc, axis=0, keepdims=True)
        p2 = jnp.maximum(zbc * jax.lax.rsqrt(v2 + EPS) * gpl_ref[0]
                         + bepl_ref[0], 0.0)
        contrib_p = wp_ref[0] * p2

        @pl.when(i == 0)
        def _():
            px_out_ref[...] = contrib_p + bp_ref[0]

        @pl.when(i != 0)
        def _():
            px_out_ref[...] = px_out_ref[...] + contrib_p


@functools.partial(jax.jit, static_argnames=("interpret",))
def _run(px, lx, pl_mat_list, W_pl, b_pl, g_pl, be_pl,
         W_lp, b_lp, g_lp, be_lp, wp2, bp2, wl2, bl2, interpret=False):
    term_blk = lambda i, r: (i, 0, 0)
    const2 = lambda i, r: (0, 0)
    vec_spec = pl.BlockSpec((1, 1, DO), term_blk)
    sca_spec = pl.BlockSpec((1, 1, 1), term_blk)
    return pl.pallas_call(
        _body,
        grid=(N_EXP, NR),
        in_specs=[
            pl.BlockSpec((1, BLK, NL), lambda i, r: (i, r, 0)),
            pl.BlockSpec((BLK, DP), lambda i, r: (r, 0)),
            pl.BlockSpec((NL, DL), const2),
            pl.BlockSpec((1, DIN, DO), term_blk),
            vec_spec,
            vec_spec,
            vec_spec,
            pl.BlockSpec((1, DIN, DO), term_blk),
            vec_spec,
            vec_spec,
            vec_spec,
            sca_spec,
            pl.BlockSpec((1, 1, 1), lambda i, r: (0, 0, 0)),
            sca_spec,
            pl.BlockSpec((1, 1, 1), lambda i, r: (0, 0, 0)),
        ],
        out_specs=[
            pl.BlockSpec((NP, DO), const2),
            pl.BlockSpec((NL, DO), const2),
        ],
        out_shape=[
            jax.ShapeDtypeStruct((NP, DO), jnp.float32),
            jax.ShapeDtypeStruct((NL, DO), jnp.float32),
        ],
        scratch_shapes=[
            pltpu.VMEM((NP, DO), jnp.float32),
            pltpu.VMEM((NL, DP), jnp.float32),
        ],
        compiler_params=pltpu.CompilerParams(
            dimension_semantics=("arbitrary", "arbitrary"),
        ),
        interpret=interpret,
    )(pl_mat_list, px, lx, W_pl, b_pl, g_pl, be_pl,
      W_lp, b_lp, g_lp, be_lp, wp2, bp2, wl2, bl2)


def kernel(px, lx, pl_mat_list, W_pl, b_pl, g_pl, be_pl,
           W_lp, b_lp, g_lp, be_lp, wp, bp, wl, bl):
    v = lambda a: a.reshape(N_EXP, 1, DO)
    px_out, lx_out = _run(px, lx, pl_mat_list,
                          W_pl, v(b_pl), v(g_pl), v(be_pl),
                          W_lp, v(b_lp), v(g_lp), v(be_lp),
                          wp.reshape(N_EXP, 1, 1), bp.reshape(1, 1, 1),
                          wl.reshape(N_EXP, 1, 1), bl.reshape(1, 1, 1))
    return (px_out, lx_out)


# speedup vs baseline: 3.5260x; 1.3936x over previous
import functools

import jax
import jax.numpy as jnp
from jax.experimental import pallas as pl
from jax.experimental.pallas import tpu as pltpu

N_EXP = 4
NP = 8192
NL = 2048
DP = 64
DL = 64
DO = 64
DIN = DP + DL
EPS = 1e-5

BLK = 1024
NR = NP // BLK


def _body(am_ref, px_ref, lx_ref,
          Wpl_ref, bpl_ref, gpl_ref, bepl_ref,
          Wlp_ref, blp_ref, glp_ref, belp_ref,
          wp_ref, bp_ref, wl_ref, bl_ref,
          px_out_ref, lx_out_ref,
          z2_ref, x1T_ref, Li_ref):
    i = pl.program_id(0)
    r = pl.program_id(1)
    nr = pl.num_programs(1)

    am = am_ref[0]
    pxb = px_ref[...]

    @pl.when(r == 0)
    def _():
        Li_ref[...] = jnp.dot(lx_ref[...], Wpl_ref[0][:DL],
                              preferred_element_type=jnp.float32)

    z2 = (jnp.dot(am, Li_ref[...], preferred_element_type=jnp.float32)
          + jnp.dot(pxb, Wpl_ref[0][DL:], preferred_element_type=jnp.float32)
          + bpl_ref[0])
    z2_ref[pl.ds(r * BLK, BLK), :] = z2

    pxw = jnp.dot(pxb, Wlp_ref[0][:DP], preferred_element_type=jnp.float32)
    partT = jax.lax.dot_general(pxw, am, (((0,), (0,)), ((), ())),
                                preferred_element_type=jnp.float32)

    @pl.when(r == 0)
    def _():
        x1T_ref[...] = partT

    @pl.when(r != 0)
    def _():
        x1T_ref[...] = x1T_ref[...] + partT

    @pl.when(r == nr - 1)
    def _():
        z1 = (x1T_ref[...].T
              + jnp.dot(lx_ref[...], Wlp_ref[0][DP:],
                        preferred_element_type=jnp.float32)
              + blp_ref[0])
        m1 = jnp.mean(z1, axis=0, keepdims=True)
        v1 = jnp.mean(z1 * z1, axis=0, keepdims=True) - m1 * m1
        s1 = glp_ref[0] * jax.lax.rsqrt(v1 + EPS)
        p1 = jnp.maximum(z1 * s1 + (belp_ref[0] - m1 * s1), 0.0)
        contrib_l = wl_ref[0] * p1

        @pl.when(i == 0)
        def _():
            lx_out_ref[...] = contrib_l + bl_ref[0]

        @pl.when(i != 0)
        def _():
            lx_out_ref[...] = lx_out_ref[...] + contrib_l

        zb = z2_ref[...]
        m2 = jnp.mean(zb, axis=0, keepdims=True)
        v2 = jnp.mean(zb * zb, axis=0, keepdims=True) - m2 * m2
        s2 = gpl_ref[0] * jax.lax.rsqrt(v2 + EPS)
        p2 = jnp.maximum(zb * s2 + (bepl_ref[0] - m2 * s2), 0.0)
        contrib_p = wp_ref[0] * p2

        @pl.when(i == 0)
        def _():
            px_out_ref[...] = contrib_p + bp_ref[0]

        @pl.when(i != 0)
        def _():
            px_out_ref[...] = px_out_ref[...] + contrib_p


@functools.partial(jax.jit, static_argnames=("interpret",))
def _run(px, lx, pl_mat_list, W_pl, b_pl, g_pl, be_pl,
         W_lp, b_lp, g_lp, be_lp, wp3, bp3, wl3, bl3, interpret=False):
    term_blk = lambda i, r: (i, 0, 0)
    const2 = lambda i, r: (0, 0)
    const3 = lambda i, r: (0, 0, 0)
    vec_spec = pl.BlockSpec((1, 1, DO), term_blk)
    sca_spec = pl.BlockSpec((1, 1, 1), term_blk)
    return pl.pallas_call(
        _body,
        grid=(N_EXP, NR),
        in_specs=[
            pl.BlockSpec((1, BLK, NL), lambda i, r: (i, r, 0)),
            pl.BlockSpec((BLK, DP), lambda i, r: (r, 0)),
            pl.BlockSpec((NL, DL), const2),
            pl.BlockSpec((1, DIN, DO), term_blk),
            vec_spec,
            vec_spec,
            vec_spec,
            pl.BlockSpec((1, DIN, DO), term_blk),
            vec_spec,
            vec_spec,
            vec_spec,
            sca_spec,
            pl.BlockSpec((1, 1, 1), const3),
            sca_spec,
            pl.BlockSpec((1, 1, 1), const3),
        ],
        out_specs=[
            pl.BlockSpec((NP, DO), const2),
            pl.BlockSpec((NL, DO), const2),
        ],
        out_shape=[
            jax.ShapeDtypeStruct((NP, DO), jnp.float32),
            jax.ShapeDtypeStruct((NL, DO), jnp.float32),
        ],
        scratch_shapes=[
            pltpu.VMEM((NP, DO), jnp.float32),
            pltpu.VMEM((DO, NL), jnp.float32),
            pltpu.VMEM((NL, DO), jnp.float32),
        ],
        compiler_params=pltpu.CompilerParams(
            dimension_semantics=("arbitrary", "arbitrary"),
        ),
        interpret=interpret,
    )(pl_mat_list, px, lx, W_pl, b_pl, g_pl, be_pl,
      W_lp, b_lp, g_lp, be_lp, wp3, bp3, wl3, bl3)


def kernel(px, lx, pl_mat_list, W_pl, b_pl, g_pl, be_pl,
           W_lp, b_lp, g_lp, be_lp, wp, bp, wl, bl):
    v = lambda a: a.reshape(N_EXP, 1, DO)
    px_out, lx_out = _run(px, lx, pl_mat_list,
                          W_pl, v(b_pl), v(g_pl), v(be_pl),
                          W_lp, v(b_lp), v(g_lp), v(be_lp),
                          wp.reshape(N_EXP, 1, 1), bp.reshape(1, 1, 1),
                          wl.reshape(N_EXP, 1, 1), bl.reshape(1, 1, 1))
    return (px_out, lx_out)
